# baseline (speedup 1.0000x reference)
import functools

import jax
import jax.numpy as jnp
from jax import lax
from jax.experimental import pallas as pl
from jax.experimental.pallas import tpu as pltpu
from jax.experimental.pallas import tpu_sc as plsc

E = 8
K = 2
D = 768
F = 3072
T = 2048
S = 2 * T
BT = 256
TP = S + E * BT
NB = TP // BT
NF = 1
FC = F // NF


_CHUNK = 512


def _router_body(x_ref, wr_ref, pos_ref, fw_ref, be_ref, br_ref, loss_ref):
    x = x_ref[...]
    wr = wr_ref[...]
    logits = lax.dot_general(x, wr, (((1,), (1,)), ((), ())),
                             preferred_element_type=jnp.float32)
    mx = jnp.max(logits, axis=1, keepdims=True)
    ex = jnp.exp(logits - mx)
    probs = ex / jnp.sum(ex, axis=1, keepdims=True)
    usage = jnp.mean(probs, axis=0, keepdims=True)
    loss_ref[...] = E * jnp.sum(usage * usage, axis=1, keepdims=True)
    iota = lax.broadcasted_iota(jnp.int32, (T, E), 1)
    ismax = logits == mx
    i1 = jnp.min(jnp.where(ismax, iota, E), axis=1, keepdims=True)
    rest = jnp.where(iota == i1, -jnp.inf, logits)
    m2 = jnp.max(rest, axis=1, keepdims=True)
    i2 = jnp.min(jnp.where(rest == m2, iota, E), axis=1, keepdims=True)
    b = jnp.exp(m2 - mx)
    fwcol = jnp.concatenate([1.0 / (1.0 + b), b / (1.0 + b)], axis=0)
    fw_ref[...] = jnp.broadcast_to(fwcol, (S, 128))

    fe = jnp.concatenate([i1, i2], axis=0)
    lanes = lax.broadcasted_iota(jnp.int32, (S, E), 1)
    oh = (fe == lanes).astype(jnp.float32)
    r2 = lax.broadcasted_iota(jnp.int32, (_CHUNK, _CHUNK), 0)
    c2 = lax.broadcasted_iota(jnp.int32, (_CHUNK, _CHUNK), 1)
    tri = (r2 >= c2).astype(jnp.float32)
    carry = jnp.zeros((1, E), jnp.float32)
    parts = []
    for k in range(S // _CHUNK):
        ohk = oh[k * _CHUNK:(k + 1) * _CHUNK]
        part = lax.dot_general(tri, ohk, (((1,), (0,)), ((), ())),
                               preferred_element_type=jnp.float32) + carry
        carry = part[_CHUNK - 1:_CHUNK]
        parts.append(part)
    cc = jnp.concatenate(parts, axis=0)
    counts = carry
    padded = jnp.floor((counts + (BT - 1)) * (1.0 / BT)) * BT
    eu = lax.broadcasted_iota(jnp.int32, (E, E), 0)
    ec = lax.broadcasted_iota(jnp.int32, (E, E), 1)
    upper = (eu < ec).astype(jnp.float32)
    aoff = lax.dot_general(padded, upper, (((1,), (0,)), ((), ())),
                           preferred_element_type=jnp.float32)
    pos_f = jnp.sum(oh * (aoff + cc - 1.0), axis=1, keepdims=True)
    pos_ref[...] = pos_f.astype(jnp.int32)
    bi = lax.broadcasted_iota(jnp.int32, (NB, E), 0).astype(jnp.float32) * BT
    be = (jnp.sum((bi >= aoff).astype(jnp.int32), axis=1, keepdims=True) - 1)
    be_ref[...] = be
    lanes_b = lax.broadcasted_iota(jnp.int32, (NB, E), 1)
    beoh = (be == lanes_b).astype(jnp.float32)
    realend = aoff + counts
    br_ref[...] = jnp.sum(beoh * (bi < realend).astype(jnp.float32),
                          axis=1, keepdims=True).astype(jnp.int32)


def _router(x, wr):
    return pl.pallas_call(
        _router_body,
        out_shape=(
            jax.ShapeDtypeStruct((S, 1), jnp.int32),
            jax.ShapeDtypeStruct((S, 128), jnp.float32),
            jax.ShapeDtypeStruct((NB, 1), jnp.int32),
            jax.ShapeDtypeStruct((NB, 1), jnp.int32),
            jax.ShapeDtypeStruct((1, 1), jnp.float32),
        ),
    )(x, wr)


_NW = 32
_DSLOT = S // _NW
_CTOK = T // _NW
_SC_MESH = plsc.VectorSubcoreMesh(core_axis_name="c", subcore_axis_name="s")


def _dispatch_body(x_hbm, pos_hbm, fw_hbm, xs_hbm, wrow_hbm,
                   ia_v, ib_v, rows_v, fwa_v, fwb_v, sem0, sem1, sem2):
    wid = lax.axis_index("s") * 2 + lax.axis_index("c")
    tbase = wid * _CTOK
    ld0 = pltpu.async_copy(pos_hbm.at[pl.ds(tbase, _CTOK)], ia_v, sem0)
    ld1 = pltpu.async_copy(pos_hbm.at[pl.ds(T + tbase, _CTOK)], ib_v, sem0)
    ld2 = pltpu.async_copy(x_hbm.at[pl.ds(tbase, _CTOK)], rows_v, sem1)
    ld3 = pltpu.async_copy(fw_hbm.at[pl.ds(tbase, _CTOK)], fwa_v, sem2)
    ld4 = pltpu.async_copy(fw_hbm.at[pl.ds(T + tbase, _CTOK)], fwb_v, sem2)
    ld0.wait()
    ld1.wait()
    ld2.wait()
    ld3.wait()
    ld4.wait()
    cp0 = pltpu.async_copy(rows_v, xs_hbm.at[ia_v], sem0)
    cp1 = pltpu.async_copy(rows_v, xs_hbm.at[ib_v], sem1)
    cp2 = pltpu.async_copy(fwa_v, wrow_hbm.at[ia_v], sem2)
    cp3 = pltpu.async_copy(fwb_v, wrow_hbm.at[ib_v], sem2)
    cp0.wait()
    cp1.wait()
    cp2.wait()
    cp3.wait()


def _dispatch(x, pos, fw):
    k = pl.kernel(
        _dispatch_body,
        mesh=_SC_MESH,
        out_type=(
            jax.ShapeDtypeStruct((TP, D), jnp.float32),
            jax.ShapeDtypeStruct((TP, 128), jnp.float32),
        ),
        scratch_types=[
            pltpu.VMEM((_CTOK,), jnp.int32),
            pltpu.VMEM((_CTOK,), jnp.int32),
            pltpu.VMEM((_CTOK, D), jnp.float32),
            pltpu.VMEM((_CTOK, 128), jnp.float32),
            pltpu.VMEM((_CTOK, 128), jnp.float32),
            pltpu.SemaphoreType.DMA,
            pltpu.SemaphoreType.DMA,
            pltpu.SemaphoreType.DMA,
        ],
    )
    return k(x, pos, fw)


def _combine_body(ys_hbm, pos_hbm, out_hbm, ia_v, ib_v, bufa_v, bufb_v,
                  sema, semb, semw):
    wid = lax.axis_index("s") * 2 + lax.axis_index("c")
    tbase = wid * _CTOK
    lda = pltpu.async_copy(pos_hbm.at[pl.ds(tbase, _CTOK)], ia_v, sema)
    ldb = pltpu.async_copy(pos_hbm.at[pl.ds(T + tbase, _CTOK)], ib_v, semb)
    lda.wait()
    ldb.wait()
    half = _CTOK // 2
    cps = []
    for ch in range(2):
        sl = pl.ds(ch * half, half)
        cps.append((pltpu.async_copy(ys_hbm.at[ia_v.at[sl]],
                                     bufa_v.at[sl], sema),
                    pltpu.async_copy(ys_hbm.at[ib_v.at[sl]],
                                     bufb_v.at[sl], semb)))

    def row(r, _):
        for c in range(D // 16):
            sl = pl.ds(c * 16, 16)
            bufa_v[r, sl] = bufa_v[r, sl] + bufb_v[r, sl]
        return 0

    wrs = []
    for ch in range(2):
        cps[ch][0].wait()
        cps[ch][1].wait()
        lax.fori_loop(ch * half, (ch + 1) * half, row, 0)
        sl = pl.ds(ch * half, half)
        wrs.append(pltpu.async_copy(bufa_v.at[sl],
                                    out_hbm.at[pl.ds(tbase + ch * half, half)],
                                    semw))
    for w in wrs:
        w.wait()


def _combine(ys, pos):
    k = pl.kernel(
        _combine_body,
        mesh=_SC_MESH,
        out_type=jax.ShapeDtypeStruct((T, D), jnp.float32),
        scratch_types=[
            pltpu.VMEM((_CTOK,), jnp.int32),
            pltpu.VMEM((_CTOK,), jnp.int32),
            pltpu.VMEM((_CTOK, D), jnp.float32),
            pltpu.VMEM((_CTOK, D), jnp.float32),
            pltpu.SemaphoreType.DMA,
            pltpu.SemaphoreType.DMA,
            pltpu.SemaphoreType.DMA,
        ],
    )
    return k(ys, pos)


def _gmm_body(be_ref, br_ref, xs_ref, w1_ref, w3_ref, w2_ref, wrow_ref,
              out_ref):
    del be_ref
    f = pl.program_id(1)

    @pl.when(br_ref[pl.program_id(0)] == 1)
    def _():
        xs = xs_ref[...]
        h = lax.dot_general(xs, w1_ref[0], (((1,), (1,)), ((), ())),
                            preferred_element_type=jnp.float32)
        g = lax.dot_general(xs, w3_ref[0], (((1,), (1,)), ((), ())),
                            preferred_element_type=jnp.float32)
        act = h * jax.nn.sigmoid(h) * g
        y = lax.dot_general(act, w2_ref[0], (((1,), (1,)), ((), ())),
                            preferred_element_type=jnp.float32)
        yw = y * wrow_ref[...][:, :1]

        @pl.when(f == 0)
        def _():
            out_ref[...] = yw

        @pl.when(f != 0)
        def _():
            out_ref[...] = out_ref[...] + yw


def _gmm(be, br, xs, w1, w3, w2, wrow):
    grid_spec = pltpu.PrefetchScalarGridSpec(
        num_scalar_prefetch=2,
        grid=(NB, NF),
        in_specs=[
            pl.BlockSpec((BT, D), lambda b, f, be_ref, br_ref: (b, 0)),
            pl.BlockSpec((1, FC, D),
                         lambda b, f, be_ref, br_ref: (be_ref[b], f, 0)),
            pl.BlockSpec((1, FC, D),
                         lambda b, f, be_ref, br_ref: (be_ref[b], f, 0)),
            pl.BlockSpec((1, D, FC),
                         lambda b, f, be_ref, br_ref: (be_ref[b], 0, f)),
            pl.BlockSpec((BT, 128), lambda b, f, be_ref, br_ref: (b, 0)),
        ],
        out_specs=pl.BlockSpec((BT, D), lambda b, f, be_ref, br_ref: (b, 0)),
    )
    return pl.pallas_call(
        _gmm_body,
        grid_spec=grid_spec,
        out_shape=jax.ShapeDtypeStruct((TP, D), jnp.float32),
        compiler_params=pltpu.CompilerParams(
            vmem_limit_bytes=100 * 1024 * 1024),
    )(be, br, xs, w1, w3, w2, wrow)


@jax.jit
def kernel(hidden_states, Wr, w1, w2, w3):
    x = hidden_states.reshape(T, D)
    pos2, fw, be2, br2, loss = _router(x, Wr)
    pos, be, br = pos2[:, 0], be2[:, 0], br2[:, 0]

    xs, wrow = _dispatch(x, pos, fw)
    ys = _gmm(be, br, xs, w1, w3, w2, wrow)
    out = _combine(ys, pos)
    return out.reshape(1, T, D), loss[0, 0]


# speedup vs baseline: 1.0109x; 1.0109x over previous
import jax
import jax.numpy as jnp
from jax import lax
from jax.experimental import pallas as pl
from jax.experimental.pallas import tpu as pltpu
from jax.experimental.pallas import tpu_sc as plsc

E = 8
K = 2
D = 768
F = 3072
T = 2048
S = 2 * T
BT = 256
TP = S + E * BT
NB = TP // BT
NF = 1
FC = F // NF


_CHUNK = 256


def _router_body(x_ref, wr_ref, pos_ref, fw_ref, be_ref, br_ref, loss_ref):
    x = x_ref[...]
    wr = wr_ref[...]
    logits = lax.dot_general(x, wr, (((1,), (1,)), ((), ())),
                             preferred_element_type=jnp.float32)
    mx = jnp.max(logits, axis=1, keepdims=True)
    ex = jnp.exp(logits - mx)
    probs = ex / jnp.sum(ex, axis=1, keepdims=True)
    usage = jnp.mean(probs, axis=0, keepdims=True)
    loss_ref[...] = E * jnp.sum(usage * usage, axis=1, keepdims=True)
    iota = lax.broadcasted_iota(jnp.int32, (T, E), 1)
    ismax = logits == mx
    i1 = jnp.min(jnp.where(ismax, iota, E), axis=1, keepdims=True)
    rest = jnp.where(iota == i1, -jnp.inf, logits)
    m2 = jnp.max(rest, axis=1, keepdims=True)
    i2 = jnp.min(jnp.where(rest == m2, iota, E), axis=1, keepdims=True)
    b = jnp.exp(m2 - mx)
    fwcol = jnp.concatenate([1.0 / (1.0 + b), b / (1.0 + b)], axis=0)
    fw_ref[...] = jnp.broadcast_to(fwcol, (S, 128))

    fe = jnp.concatenate([i1, i2], axis=0)
    lanes = lax.broadcasted_iota(jnp.int32, (S, E), 1)
    oh = (fe == lanes).astype(jnp.float32)
    r2 = lax.broadcasted_iota(jnp.int32, (_CHUNK, _CHUNK), 0)
    c2 = lax.broadcasted_iota(jnp.int32, (_CHUNK, _CHUNK), 1)
    tri = (r2 >= c2).astype(jnp.float32)
    carry = jnp.zeros((1, E), jnp.float32)
    parts = []
    for k in range(S // _CHUNK):
        ohk = oh[k * _CHUNK:(k + 1) * _CHUNK]
        part = lax.dot_general(tri, ohk, (((1,), (0,)), ((), ())),
                               preferred_element_type=jnp.float32) + carry
        carry = part[_CHUNK - 1:_CHUNK]
        parts.append(part)
    cc = jnp.concatenate(parts, axis=0)
    counts = carry
    padded = jnp.floor((counts + (BT - 1)) * (1.0 / BT)) * BT
    eu = lax.broadcasted_iota(jnp.int32, (E, E), 0)
    ec = lax.broadcasted_iota(jnp.int32, (E, E), 1)
    upper = (eu < ec).astype(jnp.float32)
    aoff = lax.dot_general(padded, upper, (((1,), (0,)), ((), ())),
                           preferred_element_type=jnp.float32)
    pos_f = jnp.sum(oh * (aoff + cc - 1.0), axis=1, keepdims=True)
    pos_ref[...] = pos_f.astype(jnp.int32)
    bi = lax.broadcasted_iota(jnp.int32, (NB, E), 0).astype(jnp.float32) * BT
    be = (jnp.sum((bi >= aoff).astype(jnp.int32), axis=1, keepdims=True) - 1)
    be_ref[...] = be
    lanes_b = lax.broadcasted_iota(jnp.int32, (NB, E), 1)
    beoh = (be == lanes_b).astype(jnp.float32)
    realend = aoff + counts
    br_ref[...] = jnp.sum(beoh * (bi < realend).astype(jnp.float32),
                          axis=1, keepdims=True).astype(jnp.int32)


def _router(x, wr):
    return pl.pallas_call(
        _router_body,
        out_shape=(
            jax.ShapeDtypeStruct((S, 1), jnp.int32),
            jax.ShapeDtypeStruct((S, 128), jnp.float32),
            jax.ShapeDtypeStruct((NB, 1), jnp.int32),
            jax.ShapeDtypeStruct((NB, 1), jnp.int32),
            jax.ShapeDtypeStruct((1, 1), jnp.float32),
        ),
    )(x, wr)


_NW = 32
_CTOK = T // _NW
_SC_MESH = plsc.VectorSubcoreMesh(core_axis_name="c", subcore_axis_name="s")


def _dispatch_body(x_hbm, pos_hbm, fw_hbm, xs_hbm, wrow_hbm,
                   ia_v, ib_v, rows_v, fwa_v, fwb_v, sem0, sem1, sem2):
    wid = lax.axis_index("s") * 2 + lax.axis_index("c")
    tbase = wid * _CTOK
    ld0 = pltpu.async_copy(pos_hbm.at[pl.ds(tbase, _CTOK)], ia_v, sem0)
    ld1 = pltpu.async_copy(pos_hbm.at[pl.ds(T + tbase, _CTOK)], ib_v, sem0)
    ld2 = pltpu.async_copy(x_hbm.at[pl.ds(tbase, _CTOK)], rows_v, sem1)
    ld3 = pltpu.async_copy(fw_hbm.at[pl.ds(tbase, _CTOK)], fwa_v, sem2)
    ld4 = pltpu.async_copy(fw_hbm.at[pl.ds(T + tbase, _CTOK)], fwb_v, sem2)
    ld0.wait()
    ld1.wait()
    ld2.wait()
    ld3.wait()
    ld4.wait()
    cp0 = pltpu.async_copy(rows_v, xs_hbm.at[ia_v], sem0)
    cp1 = pltpu.async_copy(rows_v, xs_hbm.at[ib_v], sem1)
    cp2 = pltpu.async_copy(fwa_v, wrow_hbm.at[ia_v], sem2)
    cp3 = pltpu.async_copy(fwb_v, wrow_hbm.at[ib_v], sem2)
    cp0.wait()
    cp1.wait()
    cp2.wait()
    cp3.wait()


def _dispatch(x, pos, fw):
    k = pl.kernel(
        _dispatch_body,
        mesh=_SC_MESH,
        out_type=(
            jax.ShapeDtypeStruct((TP, D), jnp.float32),
            jax.ShapeDtypeStruct((TP, 128), jnp.float32),
        ),
        scratch_types=[
            pltpu.VMEM((_CTOK,), jnp.int32),
            pltpu.VMEM((_CTOK,), jnp.int32),
            pltpu.VMEM((_CTOK, D), jnp.float32),
            pltpu.VMEM((_CTOK, 128), jnp.float32),
            pltpu.VMEM((_CTOK, 128), jnp.float32),
            pltpu.SemaphoreType.DMA,
            pltpu.SemaphoreType.DMA,
            pltpu.SemaphoreType.DMA,
        ],
    )
    return k(x, pos, fw)


def _combine_body(ys_hbm, pos_hbm, out_hbm, ia_v, ib_v, bufa_v, bufb_v,
                  sema, semb, semw):
    wid = lax.axis_index("s") * 2 + lax.axis_index("c")
    tbase = wid * _CTOK
    lda = pltpu.async_copy(pos_hbm.at[pl.ds(tbase, _CTOK)], ia_v, sema)
    ldb = pltpu.async_copy(pos_hbm.at[pl.ds(T + tbase, _CTOK)], ib_v, semb)
    lda.wait()
    ldb.wait()
    half = _CTOK // 2
    cps = []
    for ch in range(2):
        sl = pl.ds(ch * half, half)
        cps.append((pltpu.async_copy(ys_hbm.at[ia_v.at[sl]],
                                     bufa_v.at[sl], sema),
                    pltpu.async_copy(ys_hbm.at[ib_v.at[sl]],
                                     bufb_v.at[sl], semb)))

    def row(r, _):
        for c in range(D // 16):
            sl = pl.ds(c * 16, 16)
            bufa_v[r, sl] = bufa_v[r, sl] + bufb_v[r, sl]
        return 0

    wrs = []
    for ch in range(2):
        cps[ch][0].wait()
        cps[ch][1].wait()
        lax.fori_loop(ch * half, (ch + 1) * half, row, 0)
        sl = pl.ds(ch * half, half)
        wrs.append(pltpu.async_copy(bufa_v.at[sl],
                                    out_hbm.at[pl.ds(tbase + ch * half, half)],
                                    semw))
    for w in wrs:
        w.wait()


def _combine(ys, pos):
    k = pl.kernel(
        _combine_body,
        mesh=_SC_MESH,
        out_type=jax.ShapeDtypeStruct((T, D), jnp.float32),
        scratch_types=[
            pltpu.VMEM((_CTOK,), jnp.int32),
            pltpu.VMEM((_CTOK,), jnp.int32),
            pltpu.VMEM((_CTOK, D), jnp.float32),
            pltpu.VMEM((_CTOK, D), jnp.float32),
            pltpu.SemaphoreType.DMA,
            pltpu.SemaphoreType.DMA,
            pltpu.SemaphoreType.DMA,
        ],
    )
    return k(ys, pos)


def _gmm_body(be_ref, br_ref, xs_ref, w1_ref, w3_ref, w2_ref, wrow_ref,
              out_ref):
    del be_ref
    f = pl.program_id(1)

    @pl.when(br_ref[pl.program_id(0)] == 1)
    def _():
        xs = xs_ref[...]
        h = lax.dot_general(xs, w1_ref[0], (((1,), (1,)), ((), ())),
                            preferred_element_type=jnp.float32)
        g = lax.dot_general(xs, w3_ref[0], (((1,), (1,)), ((), ())),
                            preferred_element_type=jnp.float32)
        act = h * jax.nn.sigmoid(h) * g
        y = lax.dot_general(act, w2_ref[0], (((1,), (1,)), ((), ())),
                            preferred_element_type=jnp.float32)
        yw = y * wrow_ref[...][:, :1]

        @pl.when(f == 0)
        def _():
            out_ref[...] = yw

        @pl.when(f != 0)
        def _():
            out_ref[...] = out_ref[...] + yw


def _gmm(be, br, xs, w1, w3, w2, wrow):
    grid_spec = pltpu.PrefetchScalarGridSpec(
        num_scalar_prefetch=2,
        grid=(NB, NF),
        in_specs=[
            pl.BlockSpec((BT, D), lambda b, f, be_ref, br_ref: (b, 0)),
            pl.BlockSpec((1, FC, D),
                         lambda b, f, be_ref, br_ref: (be_ref[b], f, 0)),
            pl.BlockSpec((1, FC, D),
                         lambda b, f, be_ref, br_ref: (be_ref[b], f, 0)),
            pl.BlockSpec((1, D, FC),
                         lambda b, f, be_ref, br_ref: (be_ref[b], 0, f)),
            pl.BlockSpec((BT, 128), lambda b, f, be_ref, br_ref: (b, 0)),
        ],
        out_specs=pl.BlockSpec((BT, D), lambda b, f, be_ref, br_ref: (b, 0)),
    )
    return pl.pallas_call(
        _gmm_body,
        grid_spec=grid_spec,
        out_shape=jax.ShapeDtypeStruct((TP, D), jnp.float32),
        compiler_params=pltpu.CompilerParams(
            vmem_limit_bytes=100 * 1024 * 1024),
    )(be, br, xs, w1, w3, w2, wrow)


@jax.jit
def kernel(hidden_states, Wr, w1, w2, w3):
    x = hidden_states.reshape(T, D)
    pos2, fw, be2, br2, loss = _router(x, Wr)
    pos, be, br = pos2[:, 0], be2[:, 0], br2[:, 0]

    xs, wrow = _dispatch(x, pos, fw)
    ys = _gmm(be, br, xs, w1, w3, w2, wrow)
    out = _combine(ys, pos)
    return out.reshape(1, T, D), loss[0, 0]


# speedup vs baseline: 1.0113x; 1.0004x over previous
import jax
import jax.numpy as jnp
from jax import lax
from jax.experimental import pallas as pl
from jax.experimental.pallas import tpu as pltpu
from jax.experimental.pallas import tpu_sc as plsc

E = 8
K = 2
D = 768
F = 3072
T = 2048
S = 2 * T
BT = 256
TP = S + E * BT
NB = TP // BT
NF = 1
FC = F // NF


_CHUNK = 256


def _router_body(x_ref, wr_ref, pos_ref, fw_ref, be_ref, br_ref, loss_ref):
    x = x_ref[...]
    wr = wr_ref[...]
    logits = lax.dot_general(x, wr, (((1,), (1,)), ((), ())),
                             preferred_element_type=jnp.float32)
    mx = jnp.max(logits, axis=1, keepdims=True)
    ex = jnp.exp(logits - mx)
    probs = ex / jnp.sum(ex, axis=1, keepdims=True)
    usage = jnp.mean(probs, axis=0, keepdims=True)
    loss_ref[...] = E * jnp.sum(usage * usage, axis=1, keepdims=True)
    iota = lax.broadcasted_iota(jnp.int32, (T, E), 1)
    ismax = logits == mx
    i1 = jnp.min(jnp.where(ismax, iota, E), axis=1, keepdims=True)
    rest = jnp.where(iota == i1, -jnp.inf, logits)
    m2 = jnp.max(rest, axis=1, keepdims=True)
    i2 = jnp.min(jnp.where(rest == m2, iota, E), axis=1, keepdims=True)
    b = jnp.exp(m2 - mx)
    fwcol = jnp.concatenate([1.0 / (1.0 + b), b / (1.0 + b)], axis=0)
    fw_ref[...] = jnp.broadcast_to(fwcol, (S, 128))

    fe = jnp.concatenate([i1, i2], axis=0)
    lanes = lax.broadcasted_iota(jnp.int32, (S, E), 1)
    oh = (fe == lanes).astype(jnp.float32)
    r2 = lax.broadcasted_iota(jnp.int32, (_CHUNK, _CHUNK), 0)
    c2 = lax.broadcasted_iota(jnp.int32, (_CHUNK, _CHUNK), 1)
    tri = (r2 >= c2).astype(jnp.float32)
    carry = jnp.zeros((1, E), jnp.float32)
    parts = []
    for k in range(S // _CHUNK):
        ohk = oh[k * _CHUNK:(k + 1) * _CHUNK]
        part = lax.dot_general(tri, ohk, (((1,), (0,)), ((), ())),
                               preferred_element_type=jnp.float32) + carry
        carry = part[_CHUNK - 1:_CHUNK]
        parts.append(part)
    cc = jnp.concatenate(parts, axis=0)
    counts = carry
    padded = jnp.floor((counts + (BT - 1)) * (1.0 / BT)) * BT
    eu = lax.broadcasted_iota(jnp.int32, (E, E), 0)
    ec = lax.broadcasted_iota(jnp.int32, (E, E), 1)
    upper = (eu < ec).astype(jnp.float32)
    aoff = lax.dot_general(padded, upper, (((1,), (0,)), ((), ())),
                           preferred_element_type=jnp.float32)
    pos_f = jnp.sum(oh * (aoff + cc - 1.0), axis=1, keepdims=True)
    pos_ref[...] = pos_f.astype(jnp.int32)
    bi = lax.broadcasted_iota(jnp.int32, (NB, E), 0).astype(jnp.float32) * BT
    be = (jnp.sum((bi >= aoff).astype(jnp.int32), axis=1, keepdims=True) - 1)
    be_ref[...] = be
    lanes_b = lax.broadcasted_iota(jnp.int32, (NB, E), 1)
    beoh = (be == lanes_b).astype(jnp.float32)
    realend = aoff + counts
    br_ref[...] = jnp.sum(beoh * (bi < realend).astype(jnp.float32),
                          axis=1, keepdims=True).astype(jnp.int32)


def _router(x, wr):
    return pl.pallas_call(
        _router_body,
        out_shape=(
            jax.ShapeDtypeStruct((S, 1), jnp.int32),
            jax.ShapeDtypeStruct((S, 128), jnp.float32),
            jax.ShapeDtypeStruct((NB, 1), jnp.int32),
            jax.ShapeDtypeStruct((NB, 1), jnp.int32),
            jax.ShapeDtypeStruct((1, 1), jnp.float32),
        ),
    )(x, wr)


_NW = 32
_CTOK = T // _NW
_SC_MESH = plsc.VectorSubcoreMesh(core_axis_name="c", subcore_axis_name="s")


def _dispatch_body(x_hbm, pos_hbm, fw_hbm, xs_hbm, wrow_hbm,
                   ia_v, ib_v, rows_v, fwa_v, fwb_v, sem0, sem1, sem2):
    wid = lax.axis_index("s") * 2 + lax.axis_index("c")
    tbase = wid * _CTOK
    ld0 = pltpu.async_copy(pos_hbm.at[pl.ds(tbase, _CTOK)], ia_v, sem0)
    ld1 = pltpu.async_copy(pos_hbm.at[pl.ds(T + tbase, _CTOK)], ib_v, sem0)
    ld2 = pltpu.async_copy(x_hbm.at[pl.ds(tbase, _CTOK)], rows_v, sem1)
    ld3 = pltpu.async_copy(fw_hbm.at[pl.ds(tbase, _CTOK)], fwa_v, sem2)
    ld4 = pltpu.async_copy(fw_hbm.at[pl.ds(T + tbase, _CTOK)], fwb_v, sem2)
    ld0.wait()
    ld1.wait()
    ld2.wait()
    ld3.wait()
    ld4.wait()
    cp0 = pltpu.async_copy(rows_v, xs_hbm.at[ia_v], sem0)
    cp1 = pltpu.async_copy(rows_v, xs_hbm.at[ib_v], sem1)
    cp2 = pltpu.async_copy(fwa_v, wrow_hbm.at[ia_v], sem2)
    cp3 = pltpu.async_copy(fwb_v, wrow_hbm.at[ib_v], sem2)
    cp0.wait()
    cp1.wait()
    cp2.wait()
    cp3.wait()


def _dispatch(x, pos, fw):
    k = pl.kernel(
        _dispatch_body,
        mesh=_SC_MESH,
        out_type=(
            jax.ShapeDtypeStruct((TP, D), jnp.float32),
            jax.ShapeDtypeStruct((TP, 128), jnp.float32),
        ),
        scratch_types=[
            pltpu.VMEM((_CTOK,), jnp.int32),
            pltpu.VMEM((_CTOK,), jnp.int32),
            pltpu.VMEM((_CTOK, D), jnp.float32),
            pltpu.VMEM((_CTOK, 128), jnp.float32),
            pltpu.VMEM((_CTOK, 128), jnp.float32),
            pltpu.SemaphoreType.DMA,
            pltpu.SemaphoreType.DMA,
            pltpu.SemaphoreType.DMA,
        ],
    )
    return k(x, pos, fw)


def _combine_body(ys_hbm, pos_hbm, out_hbm, ia_v, ib_v, bufa_v, bufb_v,
                  sema, semb, semw):
    wid = lax.axis_index("s") * 2 + lax.axis_index("c")
    tbase = wid * _CTOK
    lda = pltpu.async_copy(pos_hbm.at[pl.ds(tbase, _CTOK)], ia_v, sema)
    ldb = pltpu.async_copy(pos_hbm.at[pl.ds(T + tbase, _CTOK)], ib_v, semb)
    lda.wait()
    ldb.wait()
    half = _CTOK // 2
    cps = []
    for ch in range(2):
        sl = pl.ds(ch * half, half)
        cps.append((pltpu.async_copy(ys_hbm.at[ia_v.at[sl]],
                                     bufa_v.at[sl], sema),
                    pltpu.async_copy(ys_hbm.at[ib_v.at[sl]],
                                     bufb_v.at[sl], semb)))

    def row(r, _):
        for c in range(D // 16):
            sl = pl.ds(c * 16, 16)
            bufa_v[r, sl] = bufa_v[r, sl] + bufb_v[r, sl]
        return 0

    wrs = []
    for ch in range(2):
        cps[ch][0].wait()
        cps[ch][1].wait()
        lax.fori_loop(ch * half, (ch + 1) * half, row, 0)
        sl = pl.ds(ch * half, half)
        wrs.append(pltpu.async_copy(bufa_v.at[sl],
                                    out_hbm.at[pl.ds(tbase + ch * half, half)],
                                    semw))
    for w in wrs:
        w.wait()


def _combine(ys, pos):
    k = pl.kernel(
        _combine_body,
        mesh=_SC_MESH,
        out_type=jax.ShapeDtypeStruct((T, D), jnp.float32),
        scratch_types=[
            pltpu.VMEM((_CTOK,), jnp.int32),
            pltpu.VMEM((_CTOK,), jnp.int32),
            pltpu.VMEM((_CTOK, D), jnp.float32),
            pltpu.VMEM((_CTOK, D), jnp.float32),
            pltpu.SemaphoreType.DMA,
            pltpu.SemaphoreType.DMA,
            pltpu.SemaphoreType.DMA,
        ],
    )
    return k(ys, pos)


def _gmm_body(be_ref, br_ref, xs_ref, w1_ref, w3_ref, w2_ref, wrow_ref,
              out_ref):
    del be_ref

    @pl.when(br_ref[pl.program_id(0)] == 1)
    def _():
        xs = xs_ref[...]
        h = lax.dot_general(xs, w1_ref[0], (((1,), (1,)), ((), ())),
                            preferred_element_type=jnp.float32)
        g = lax.dot_general(xs, w3_ref[0], (((1,), (1,)), ((), ())),
                            preferred_element_type=jnp.float32)
        act = h * jax.nn.sigmoid(h) * g
        y = lax.dot_general(act, w2_ref[0], (((1,), (1,)), ((), ())),
                            preferred_element_type=jnp.float32)
        out_ref[...] = y * wrow_ref[...][:, :1]


def _gmm(be, br, xs, w1, w3, w2, wrow):
    grid_spec = pltpu.PrefetchScalarGridSpec(
        num_scalar_prefetch=2,
        grid=(NB,),
        in_specs=[
            pl.BlockSpec((BT, D), lambda b, be_ref, br_ref: (b, 0)),
            pl.BlockSpec((1, F, D),
                         lambda b, be_ref, br_ref: (be_ref[b], 0, 0)),
            pl.BlockSpec((1, F, D),
                         lambda b, be_ref, br_ref: (be_ref[b], 0, 0)),
            pl.BlockSpec((1, D, F),
                         lambda b, be_ref, br_ref: (be_ref[b], 0, 0)),
            pl.BlockSpec((BT, 128), lambda b, be_ref, br_ref: (b, 0)),
        ],
        out_specs=pl.BlockSpec((BT, D), lambda b, be_ref, br_ref: (b, 0)),
    )
    return pl.pallas_call(
        _gmm_body,
        grid_spec=grid_spec,
        out_shape=jax.ShapeDtypeStruct((TP, D), jnp.float32),
        compiler_params=pltpu.CompilerParams(
            vmem_limit_bytes=100 * 1024 * 1024),
    )(be, br, xs, w1, w3, w2, wrow)


@jax.jit
def kernel(hidden_states, Wr, w1, w2, w3):
    x = hidden_states.reshape(T, D)
    pos2, fw, be2, br2, loss = _router(x, Wr)
    pos, be, br = pos2[:, 0], be2[:, 0], br2[:, 0]

    xs, wrow = _dispatch(x, pos, fw)
    ys = _gmm(be, br, xs, w1, w3, w2, wrow)
    out = _combine(ys, pos)
    return out.reshape(1, T, D), loss[0, 0]
